# baseline (speedup 1.0000x reference)
import functools

import jax
import jax.numpy as jnp
from jax import lax
from jax.experimental import pallas as pl
from jax.experimental.pallas import tpu as pltpu
from jax.experimental.pallas import tpu_sc as plsc

B_BLK = 256
NW = 32
CHUNK = 128


CW = 64


def _trunc_bf16(x):
    xi = lax.bitcast_convert_type(x, jnp.uint32)
    return lax.bitcast_convert_type(xi & jnp.uint32(0xFFFF0000), jnp.float32)


def _split3(x):
    h = _trunc_bf16(x)
    r = x - h
    m = _trunc_bf16(r)
    l = r - m
    return h.astype(jnp.bfloat16), m.astype(jnp.bfloat16), l.astype(jnp.bfloat16)


def _assign_body(a_ref, cb_ref, idx_ref):
    P, _, K = cb_ref.shape
    iota_f = lax.broadcasted_iota(jnp.int32, (B_BLK, K), 1).astype(jnp.float32)

    def _argmax_store(p, s):
        m = jnp.max(s, axis=1, keepdims=True)
        idx_ref[:, p] = jnp.min((m - s) + iota_f, axis=1)

    prev = None
    for p in range(P):
        s = lax.dot_general(a_ref[:, p * CW:(p + 1) * CW], cb_ref[p],
                            (((1,), (0,)), ((), ())),
                            preferred_element_type=jnp.float32)
        if prev is not None:
            _argmax_store(*prev)
        prev = (p, s)
    _argmax_store(*prev)


def _sc_gather(table, idx3, d):
    nw, c, chunk = idx3.shape
    b_per_w = c * chunk
    mesh = plsc.VectorSubcoreMesh(core_axis_name="c", subcore_axis_name="s")

    @functools.partial(
        pl.kernel,
        out_type=jax.ShapeDtypeStruct((nw * b_per_w, d), jnp.float32),
        mesh=mesh,
        scratch_types=[
            pltpu.VMEM((c, chunk), jnp.int32),
            pltpu.VMEM((b_per_w, d), jnp.float32),
            pltpu.SemaphoreType.DMA,
        ],
        compiler_params=pltpu.CompilerParams(use_tc_tiling_on_sc=False),
    )
    def gather_kernel(table_hbm, idx_hbm, out_hbm, idx_v, rows_v, sem):
        wid = lax.axis_index("s") * 2 + lax.axis_index("c")
        pltpu.sync_copy(idx_hbm.at[wid], idx_v)

        def step(s_, carry):
            copies = [
                pltpu.async_copy(
                    table_hbm.at[idx_v.at[s_ * 8 + i]],
                    rows_v.at[pl.ds((s_ * 8 + i) * chunk, chunk)],
                    sem,
                )
                for i in range(8)
            ]
            for cp in copies:
                cp.wait()
            return carry

        lax.fori_loop(0, c // 8, step, 0)
        pltpu.sync_copy(rows_v, out_hbm.at[pl.ds(wid * b_per_w, b_per_w)])

    return gather_kernel(table, idx3)


def kernel(vecs, codebook):
    B, E = vecs.shape
    P, K, D = codebook.shape

    vh, vm, vl = _split3(vecs.reshape(B, P, D))
    ch, cm, cl = _split3(codebook * jnp.float32(2.0 ** 40))
    cnh, cnm, cnl = _split3(-0.5 * jnp.float32(2.0 ** 40)
                            * jnp.sum(codebook * codebook, axis=-1))

    va = jnp.stack([vh, vh, vm, vh, vm, vl], axis=2).reshape(B, P, 6 * D)
    a_full = jnp.concatenate(
        [va, jnp.ones((B, P, 3), jnp.bfloat16),
         jnp.zeros((B, P, CW - 6 * D - 3), jnp.bfloat16)],
        axis=-1).reshape(B, P * CW)

    cb6 = jnp.stack([ch, cm, ch, cl, cm, ch], axis=1)
    cb6 = jnp.transpose(cb6, (0, 1, 3, 2)).reshape(P, 6 * D, K)
    bias = jnp.stack([cnh, cnm, cnl], axis=1)
    cb_full = jnp.concatenate(
        [cb6, bias, jnp.zeros((P, CW - 6 * D - 3, K), jnp.bfloat16)], axis=1)

    flat_idx = pl.pallas_call(
        _assign_body,
        grid=(B // B_BLK,),
        in_specs=[
            pl.BlockSpec((B_BLK, P * CW), lambda j: (j, 0)),
            pl.BlockSpec((P, CW, K), lambda j: (0, 0, 0)),
        ],
        out_specs=pl.BlockSpec((B_BLK, P), lambda j: (j, 0)),
        out_shape=jax.ShapeDtypeStruct((B, P), jnp.float32),
    )(a_full, cb_full)
    flat_idx = (flat_idx + (K * jnp.arange(P, dtype=jnp.float32))[None, :]).astype(jnp.int32)

    table = codebook.reshape(P * K, D)
    idx3 = flat_idx.reshape(NW, (B * P) // (NW * CHUNK), CHUNK)
    rows = _sc_gather(table, idx3, D)
    return rows.reshape(B, P * D)


# speedup vs baseline: 2.2929x; 2.2929x over previous
import functools

import jax
import jax.numpy as jnp
from jax import lax
from jax.experimental import pallas as pl
from jax.experimental.pallas import tpu as pltpu
from jax.experimental.pallas import tpu_sc as plsc

B_BLK = 256
NW = 32
CHUNK = 128


CW = 64


def _trunc_bf16(x):
    xi = lax.bitcast_convert_type(x, jnp.uint32)
    return lax.bitcast_convert_type(xi & jnp.uint32(0xFFFF0000), jnp.float32)


def _split3(x):
    h = _trunc_bf16(x)
    r = x - h
    m = _trunc_bf16(r)
    l = r - m
    return h.astype(jnp.bfloat16), m.astype(jnp.bfloat16), l.astype(jnp.bfloat16)


def _assign_body(v_ref, cb_ref, idx_ref):
    P, K, _ = cb_ref.shape
    D = v_ref.shape[1] // P
    iota_f = lax.broadcasted_iota(jnp.int32, (K, B_BLK), 0).astype(jnp.float32)

    vt = jnp.transpose(v_ref[:])
    vh = _trunc_bf16(vt)
    r = vt - vh
    vm = _trunc_bf16(r)
    vl = r - vm
    const_rows = jnp.concatenate(
        [jnp.ones((3, B_BLK), jnp.float32), jnp.zeros((13, B_BLK), jnp.float32)],
        axis=0)

    def _argmax_store(p, s):
        m = jnp.max(s, axis=0, keepdims=True)
        idx_ref[p, :] = jnp.min((m - s) + iota_f, axis=0)

    prev = None
    for p in range(P):
        sl = slice(p * D, (p + 1) * D)
        rhs = jnp.concatenate(
            [vh[sl], vh[sl], vm[sl], vh[sl], vm[sl], vl[sl], const_rows],
            axis=0).astype(jnp.bfloat16)
        s = lax.dot_general(cb_ref[p], rhs, (((1,), (0,)), ((), ())),
                            preferred_element_type=jnp.float32)
        if prev is not None:
            _argmax_store(*prev)
        prev = (p, s)
    _argmax_store(*prev)


def _sc_gather(table, idx3, d):
    nw, c, chunk = idx3.shape
    b_per_w = c * chunk
    mesh = plsc.VectorSubcoreMesh(core_axis_name="c", subcore_axis_name="s")

    @functools.partial(
        pl.kernel,
        out_type=jax.ShapeDtypeStruct((nw * b_per_w, d), jnp.float32),
        mesh=mesh,
        scratch_types=[
            pltpu.VMEM((c, chunk), jnp.int32),
            pltpu.VMEM((b_per_w, d), jnp.float32),
            pltpu.SemaphoreType.DMA,
        ],
        compiler_params=pltpu.CompilerParams(use_tc_tiling_on_sc=False),
    )
    def gather_kernel(table_hbm, idx_hbm, out_hbm, idx_v, rows_v, sem):
        wid = lax.axis_index("s") * 2 + lax.axis_index("c")
        pltpu.sync_copy(idx_hbm.at[wid], idx_v)

        def step(s_, carry):
            copies = [
                pltpu.async_copy(
                    table_hbm.at[idx_v.at[s_ * 8 + i]],
                    rows_v.at[pl.ds((s_ * 8 + i) * chunk, chunk)],
                    sem,
                )
                for i in range(8)
            ]
            for cp in copies:
                cp.wait()
            return carry

        lax.fori_loop(0, c // 8, step, 0)
        pltpu.sync_copy(rows_v, out_hbm.at[pl.ds(wid * b_per_w, b_per_w)])

    return gather_kernel(table, idx3)


def kernel(vecs, codebook):
    B, E = vecs.shape
    P, K, D = codebook.shape

    ch, cm, cl = _split3(codebook * jnp.float32(2.0 ** 40))
    cnh, cnm, cnl = _split3(-0.5 * jnp.float32(2.0 ** 40)
                            * jnp.sum(codebook * codebook, axis=-1))

    cb6 = jnp.stack([ch, cm, ch, cl, cm, ch], axis=2).reshape(P, K, 6 * D)
    bias = jnp.stack([cnh, cnm, cnl], axis=-1)
    cb_full = jnp.concatenate(
        [cb6, bias, jnp.zeros((P, K, CW - 6 * D - 3), jnp.bfloat16)], axis=-1)

    idx_t = pl.pallas_call(
        _assign_body,
        grid=(B // B_BLK,),
        in_specs=[
            pl.BlockSpec((B_BLK, E), lambda j: (j, 0)),
            pl.BlockSpec((P, K, CW), lambda j: (0, 0, 0)),
        ],
        out_specs=pl.BlockSpec((P, B_BLK), lambda j: (0, j)),
        out_shape=jax.ShapeDtypeStruct((P, B), jnp.float32),
    )(vecs, cb_full)
    flat_idx = (jnp.transpose(idx_t)
                + (K * jnp.arange(P, dtype=jnp.float32))[None, :]).astype(jnp.int32)

    table = codebook.reshape(P * K, D)
    idx3 = flat_idx.reshape(NW, (B * P) // (NW * CHUNK), CHUNK)
    rows = _sc_gather(table, idx3, D)
    return rows.reshape(B, P * D)
